# baseline (speedup 1.0000x reference)
import functools

import jax
import jax.numpy as jnp
from jax import lax
from jax.experimental import pallas as pl
from jax.experimental.pallas import tpu as pltpu
from jax.experimental.pallas import tpu_sc as plsc

_VOCAB = 100000
_CTX = 2048
_DIM = 1024
_BATCH = 4

_NC = 2
_NS = 16
_NW = _NC * _NS
_PW = _CTX // _NW
_SUB = 32
_QTR = _SUB // 4
_HALF = _SUB // 2
_STEPS = 2 * _BATCH
_NBUF = 2
_LANES = 16


def _body(x_hbm, tok_hbm, pos_hbm, out_hbm, idx_v, pos_v, rows0, rows1,
          *sems):
    rows = [rows0, rows1]
    gsem = [sems[0:4], sems[4:8]]
    osem = [sems[8:12], sems[12:16]]
    psem = sems[16]

    wid = lax.axis_index("s") * _NC + lax.axis_index("c")
    p0 = wid * _PW

    gd = {}
    od = {}

    def gather(s, q):
        h, b = divmod(s, _BATCH)
        gd[s, q] = pltpu.async_copy(
            tok_hbm.at[idx_v.at[b, pl.ds(h * _SUB + q * _QTR, _QTR)]],
            rows[s % _NBUF].at[pl.ds(q * _QTR, _QTR)],
            gsem[s % _NBUF][q])

    def outcopy(s, q):
        h, b = divmod(s, _BATCH)
        od[s, q] = pltpu.async_copy(
            rows[s % _NBUF].at[pl.ds(q * _QTR, _QTR)],
            out_hbm.at[b, pl.ds(p0 + h * _SUB + q * _QTR, _QTR)],
            osem[s % _NBUF][q])

    def stage_pos(h):
        return pltpu.async_copy(
            pos_hbm.at[pl.ds(p0 + h * _SUB, _SUB)], pos_v, psem)

    def add_pos(s, half):
        buf = rows[s % _NBUF]

        def add_row(r, _):
            for d in range(_DIM // _LANES):
                sl = pl.ds(d * _LANES, _LANES)
                plsc.addupdate(buf.at[r, sl], pos_v[r, sl])
            return 0

        lax.fori_loop(half * _HALF, (half + 1) * _HALF, add_row, 0)

    pltpu.sync_copy(x_hbm.at[0, pl.ds(p0, _PW)], idx_v.at[0])
    for q in range(4):
        gather(0, q)
    pltpu.sync_copy(x_hbm.at[1, pl.ds(p0, _PW)], idx_v.at[1])
    for q in range(4):
        gather(1, q)
    pd = stage_pos(0)
    pltpu.sync_copy(x_hbm.at[2, pl.ds(p0, _PW)], idx_v.at[2])
    pltpu.sync_copy(x_hbm.at[3, pl.ds(p0, _PW)], idx_v.at[3])
    for s in range(_STEPS):
        more = s + 1 < _STEPS
        for half in range(2):
            for q in (2 * half, 2 * half + 1):
                if s >= 1 and more:
                    od[s - 1, q].wait()
                    gather(s + 1, q)
                gd[s, q].wait()
            if half == 0 and (s == 0 or s == _BATCH):
                pd.wait()
            add_pos(s, half)
            outcopy(s, 2 * half)
            outcopy(s, 2 * half + 1)
        if s == _BATCH - 1:
            pd = stage_pos(1)
    for s in (_STEPS - 2, _STEPS - 1):
        for q in range(4):
            od[s, q].wait()


def kernel(x, token_table, pos_table):
    mesh = plsc.VectorSubcoreMesh(core_axis_name="c", subcore_axis_name="s")
    run = functools.partial(
        pl.kernel,
        mesh=mesh,
        out_type=jax.ShapeDtypeStruct((_BATCH, _CTX, _DIM), jnp.float32),
        scratch_types=(
            [pltpu.VMEM((_BATCH, _PW), jnp.int32),
             pltpu.VMEM((_SUB, _DIM), jnp.float32),
             pltpu.VMEM((_SUB, _DIM), jnp.float32),
             pltpu.VMEM((_SUB, _DIM), jnp.float32)]
            + [pltpu.SemaphoreType.DMA] * 17
        ),
    )(_body)
    return run(x, token_table, pos_table)
